# Initial kernel scaffold; baseline (speedup 1.0000x reference)
import jax
import jax.numpy as jnp
from jax.experimental import pallas as pl


def kernel(x, real_w, imag_w):
    raise NotImplementedError("write your pallas kernel here")


# speedup vs baseline: 3.3631x; 3.3631x over previous
import functools

import jax
import jax.numpy as jnp
from jax import lax
from jax.experimental import pallas as pl
from jax.experimental.pallas import tpu as pltpu
from jax.experimental.pallas import tpu_sc as plsc

NUM_EMB = 100000
D = 128
B = 16384
H = 50
FLAT = B * H
NC = 2
NS = 16
NW = NC * NS
PER_W = FLAT // NW
CHUNK = 128
CHUNKS_PER_W = PER_W // CHUNK


def _emb_body(x_hbm, real_hbm, imag_hbm, real_out, imag_out,
              idx_v, rrow, irow, rsem, isem):
    wid = lax.axis_index("s") * NC + lax.axis_index("c")
    base_chunk = wid * CHUNKS_PER_W
    pltpu.sync_copy(x_hbm.at[pl.ds(base_chunk, CHUNKS_PER_W)], idx_v)

    def body(j, carry):
        row0 = (base_chunk + j) * CHUNK
        pltpu.async_copy(real_hbm.at[idx_v.at[j]], rrow, rsem).wait()
        pltpu.sync_copy(rrow, real_out.at[pl.ds(row0, CHUNK)])
        pltpu.async_copy(imag_hbm.at[idx_v.at[j]], irow, isem).wait()
        pltpu.sync_copy(irow, imag_out.at[pl.ds(row0, CHUNK)])
        return carry

    lax.fori_loop(0, CHUNKS_PER_W, body, 0)


@jax.jit
def _run(x2d, real_w, imag_w):
    mesh = plsc.VectorSubcoreMesh(core_axis_name="c", subcore_axis_name="s")
    f = functools.partial(
        pl.kernel,
        out_type=[
            jax.ShapeDtypeStruct((FLAT, D), jnp.float32),
            jax.ShapeDtypeStruct((FLAT, D), jnp.float32),
        ],
        mesh=mesh,
        scratch_types=[
            pltpu.VMEM((CHUNKS_PER_W, CHUNK), jnp.int32),
            pltpu.VMEM((CHUNK, D), jnp.float32),
            pltpu.VMEM((CHUNK, D), jnp.float32),
            pltpu.SemaphoreType.DMA,
            pltpu.SemaphoreType.DMA,
        ],
    )(_emb_body)
    return f(x2d, real_w, imag_w)


def kernel(x, real_w, imag_w):
    x2d = x.reshape(FLAT // CHUNK, CHUNK).astype(jnp.int32)
    real_flat, imag_flat = _run(x2d, real_w, imag_w)
    return (real_flat.reshape(B, H, D), imag_flat.reshape(B, H, D))


# speedup vs baseline: 3.8421x; 1.1424x over previous
import functools

import jax
import jax.numpy as jnp
from jax import lax
from jax.experimental import pallas as pl
from jax.experimental.pallas import tpu as pltpu
from jax.experimental.pallas import tpu_sc as plsc

NUM_EMB = 100000
D = 128
B = 16384
H = 50
FLAT = B * H
NC = 2
NS = 16
NW = NC * NS
PER_W = FLAT // NW
CHUNK = 128
CHUNKS_PER_W = PER_W // CHUNK
NBUF = 2
GROUPS = CHUNKS_PER_W // NBUF


def _emb_body(x_hbm, real_hbm, imag_hbm, real_out, imag_out,
              idx_v, rbuf, ibuf, rgsem, igsem, rwsem, iwsem):
    wid = lax.axis_index("s") * NC + lax.axis_index("c")
    base_chunk = wid * CHUNKS_PER_W
    pltpu.sync_copy(x_hbm.at[pl.ds(base_chunk, CHUNKS_PER_W)], idx_v)

    def gather(j, b):
        pltpu.async_copy(real_hbm.at[idx_v.at[j]], rbuf.at[b], rgsem.at[b])
        pltpu.async_copy(imag_hbm.at[idx_v.at[j]], ibuf.at[b], igsem.at[b])

    for b in range(NBUF):
        gather(b, b)

    def body(g, carry):
        base = g * NBUF
        for b in range(NBUF):
            j = base + b
            row0 = (base_chunk + j) * CHUNK
            pltpu.make_async_copy(real_hbm.at[idx_v.at[j]], rbuf.at[b],
                                  rgsem.at[b]).wait()
            pltpu.async_copy(rbuf.at[b], real_out.at[pl.ds(row0, CHUNK)],
                             rwsem.at[b])
            pltpu.make_async_copy(imag_hbm.at[idx_v.at[j]], ibuf.at[b],
                                  igsem.at[b]).wait()
            pltpu.async_copy(ibuf.at[b], imag_out.at[pl.ds(row0, CHUNK)],
                             iwsem.at[b])

        @pl.when(g < GROUPS - 1)
        def _():
            for b in range(NBUF):
                j = base + NBUF + b
                row0 = (base_chunk + base - NBUF + b) * CHUNK
                pltpu.make_async_copy(rbuf.at[b],
                                      real_out.at[pl.ds(row0, CHUNK)],
                                      rwsem.at[b]).wait()
                pltpu.make_async_copy(ibuf.at[b],
                                      imag_out.at[pl.ds(row0, CHUNK)],
                                      iwsem.at[b]).wait()
                gather(j, b)

        return carry

    lax.fori_loop(0, GROUPS, body, 0)

    last = GROUPS - 1
    for b in range(NBUF):
        row0 = (base_chunk + last * NBUF + b) * CHUNK
        pltpu.make_async_copy(rbuf.at[b], real_out.at[pl.ds(row0, CHUNK)],
                              rwsem.at[b]).wait()
        pltpu.make_async_copy(ibuf.at[b], imag_out.at[pl.ds(row0, CHUNK)],
                              iwsem.at[b]).wait()


@jax.jit
def _run(x2d, real_w, imag_w):
    mesh = plsc.VectorSubcoreMesh(core_axis_name="c", subcore_axis_name="s")
    f = functools.partial(
        pl.kernel,
        out_type=[
            jax.ShapeDtypeStruct((FLAT, D), jnp.float32),
            jax.ShapeDtypeStruct((FLAT, D), jnp.float32),
        ],
        mesh=mesh,
        scratch_types=[
            pltpu.VMEM((CHUNKS_PER_W, CHUNK), jnp.int32),
            pltpu.VMEM((NBUF, CHUNK, D), jnp.float32),
            pltpu.VMEM((NBUF, CHUNK, D), jnp.float32),
            pltpu.SemaphoreType.DMA((NBUF,)),
            pltpu.SemaphoreType.DMA((NBUF,)),
            pltpu.SemaphoreType.DMA((NBUF,)),
            pltpu.SemaphoreType.DMA((NBUF,)),
        ],
    )(_emb_body)
    return f(x2d, real_w, imag_w)


def kernel(x, real_w, imag_w):
    x2d = x.reshape(FLAT // CHUNK, CHUNK).astype(jnp.int32)
    real_flat, imag_flat = _run(x2d, real_w, imag_w)
    return (real_flat.reshape(B, H, D), imag_flat.reshape(B, H, D))
